# baseline (speedup 1.0000x reference)
import jax
import jax.numpy as jnp
from jax import lax
from jax.experimental import pallas as pl
from jax.experimental.pallas import tpu as pltpu
from jax.experimental.pallas import tpu_sc as plsc

_N_E = 8192
_E_DIM = 32
_BETA = 0.25
_COLS = 512
_CCH = 4096


def _tc_argmin_body(z_ref, wb2_ref, idx_ref, loss_ref):
    g = pl.program_id(0)
    zc = z_ref[0]

    zn = zc[0:1, :] * zc[0:1, :]
    for i in range(1, _E_DIM):
        zn = zn + zc[i:i + 1, :] * zc[i:i + 1, :]

    zcb = zc.astype(jnp.bfloat16)

    acc_v = jnp.full((1, _COLS), jnp.inf, jnp.float32)
    acc_i = jnp.zeros((1, _COLS), jnp.int32)
    loss_v = jnp.zeros((1, _COLS), jnp.float32)
    for k in range(_N_E // _CCH):
        w2k = wb2_ref[pl.ds(k * _CCH, _CCH), :]
        mm2 = lax.dot_general(
            w2k, zcb,
            dimension_numbers=(((1,), (0,)), ((), ())),
            preferred_element_type=jnp.float32,
        )
        dk = zn - mm2
        cmin = jnp.min(dk, axis=0, keepdims=True)
        ii = lax.broadcasted_iota(jnp.int32, dk.shape, 0) \
            .astype(jnp.float32) + jnp.float32(2.0**23)
        cf = jnp.min(jnp.where(dk == cmin, ii, jnp.float32(2.0**24)),
                     axis=0, keepdims=True)
        cidx = cf.astype(jnp.int32) - jnp.int32(2**23) + k * _CCH
        win = cmin < acc_v
        acc_i = jnp.where(win, cidx, acc_i)
        loss_v = jnp.where(win, cmin, loss_v)
        acc_v = jnp.where(win, cmin, acc_v).astype(jnp.bfloat16) \
                                           .astype(jnp.float32)
    idx_ref[...] = acc_i.reshape(1, 1, _COLS)

    @pl.when(g == 0)
    def _():
        loss_ref[0, 0] = 0.0

    loss_ref[0, 0] += jnp.sum(loss_v)


def _tc_argmin(zr, Wb2):
    b = zr.shape[0]
    hw = zr.shape[2]
    steps = (b * hw) // _COLS
    per_b = hw // _COLS
    return pl.pallas_call(
        _tc_argmin_body,
        grid=(steps,),
        in_specs=[
            pl.BlockSpec((1, _E_DIM, _COLS),
                         lambda g: (g // per_b, 0, g % per_b)),
            pl.BlockSpec((_N_E, _E_DIM), lambda g: (0, 0)),
        ],
        out_specs=[
            pl.BlockSpec((1, 1, _COLS), lambda g: (g, 0, 0)),
            pl.BlockSpec(memory_space=pltpu.SMEM),
        ],
        out_shape=[
            jax.ShapeDtypeStruct((steps, 1, _COLS), jnp.int32),
            jax.ShapeDtypeStruct((1, 1), jnp.float32),
        ],
    )(zr, Wb2)


_NW = 32
_ROWS_PER_W = 512
_CHUNK = 128
_NCHUNK = _ROWS_PER_W // _CHUNK


def _sc_gather_body(w_hbm, idx_hbm, out_hbm, idx_v, rows_v, sem):
    wid = lax.axis_index("s") * 2 + lax.axis_index("c")
    base = wid * _ROWS_PER_W
    pltpu.sync_copy(idx_hbm.at[wid], idx_v)
    cps = [
        pltpu.async_copy(w_hbm.at[idx_v.at[j]],
                         rows_v.at[pl.ds(j * _CHUNK, _CHUNK)], sem)
        for j in range(_NCHUNK)
    ]
    for cp in cps:
        cp.wait()
    pltpu.sync_copy(rows_v, out_hbm.at[pl.ds(base, _ROWS_PER_W)])


def _sc_gather(W, idx3):
    gk = pl.kernel(
        _sc_gather_body,
        out_type=jax.ShapeDtypeStruct((_NW * _ROWS_PER_W, _E_DIM),
                                      jnp.float32),
        mesh=plsc.VectorSubcoreMesh(core_axis_name="c",
                                    subcore_axis_name="s"),
        scratch_types=[
            pltpu.VMEM((_NCHUNK, _CHUNK), jnp.int32),
            pltpu.VMEM((_ROWS_PER_W, _E_DIM), jnp.float32),
            pltpu.SemaphoreType.DMA,
        ],
        compiler_params=pltpu.CompilerParams(use_tc_tiling_on_sc=False),
    )
    return gk(W, idx3)


def kernel(z, W):
    b, c, h, w = z.shape
    n = b * h * w
    zr = z.reshape(b, c, h * w)
    Wb2 = (2.0 * W).astype(jnp.bfloat16)

    idx2, loss_sum = _tc_argmin(zr, Wb2)
    idx_flat = idx2.reshape(n)
    idx3 = idx2.reshape(_NW, _NCHUNK, _CHUNK)

    zq_flat = _sc_gather(W, idx3)

    zq = zq_flat.reshape(b, h, w, c)
    z_q_out = jnp.transpose(zq, (0, 3, 1, 2))

    m = loss_sum[0, 0] / jnp.float32(n * c)
    loss = m + _BETA * m

    z_indices = idx_flat.reshape(b, 1, h, w)
    return (z_q_out, loss, idx_flat, z_indices)


# speedup vs baseline: 1.0706x; 1.0706x over previous
import jax
import jax.numpy as jnp
from jax import lax
from jax.experimental import pallas as pl
from jax.experimental.pallas import tpu as pltpu
from jax.experimental.pallas import tpu_sc as plsc

_N_E = 8192
_E_DIM = 32
_BETA = 0.25
_COLS = 1024
_CCH = 4096


def _tc_argmin_body(z_ref, wb2_ref, idx_ref, loss_ref):
    g = pl.program_id(0)
    zc = z_ref[0]

    zn = zc[0:1, :] * zc[0:1, :]
    for i in range(1, _E_DIM):
        zn = zn + zc[i:i + 1, :] * zc[i:i + 1, :]

    zcb = zc.astype(jnp.bfloat16)

    acc_v = jnp.full((1, _COLS), jnp.inf, jnp.float32)
    acc_i = jnp.zeros((1, _COLS), jnp.int32)
    loss_v = jnp.zeros((1, _COLS), jnp.float32)
    for k in range(_N_E // _CCH):
        w2k = wb2_ref[pl.ds(k * _CCH, _CCH), :]
        mm2 = lax.dot_general(
            w2k, zcb,
            dimension_numbers=(((1,), (0,)), ((), ())),
            preferred_element_type=jnp.float32,
        )
        dk = zn - mm2
        cmin = jnp.min(dk, axis=0, keepdims=True)
        ii = lax.broadcasted_iota(jnp.int32, dk.shape, 0)
        cidx = jnp.min(jnp.where(dk == cmin, ii, jnp.int32(2**30)),
                       axis=0, keepdims=True) + k * _CCH
        win = cmin < acc_v
        acc_i = jnp.where(win, cidx, acc_i)
        loss_v = jnp.where(win, cmin, loss_v)
        acc_v = jnp.where(win, cmin, acc_v).astype(jnp.bfloat16) \
                                           .astype(jnp.float32)
    idx_ref[...] = acc_i.reshape(1, 1, _COLS)

    @pl.when(g == 0)
    def _():
        loss_ref[0, 0] = 0.0

    loss_ref[0, 0] += jnp.sum(loss_v)


def _tc_argmin(zr, Wb2):
    b = zr.shape[0]
    hw = zr.shape[2]
    steps = (b * hw) // _COLS
    per_b = hw // _COLS
    return pl.pallas_call(
        _tc_argmin_body,
        grid=(steps,),
        in_specs=[
            pl.BlockSpec((1, _E_DIM, _COLS),
                         lambda g: (g // per_b, 0, g % per_b)),
            pl.BlockSpec((_N_E, _E_DIM), lambda g: (0, 0)),
        ],
        out_specs=[
            pl.BlockSpec((1, 1, _COLS), lambda g: (g, 0, 0)),
            pl.BlockSpec(memory_space=pltpu.SMEM),
        ],
        out_shape=[
            jax.ShapeDtypeStruct((steps, 1, _COLS), jnp.int32),
            jax.ShapeDtypeStruct((1, 1), jnp.float32),
        ],
    )(zr, Wb2)


_NW = 32
_ROWS_PER_W = 512
_CHUNK = 128
_NCHUNK = _ROWS_PER_W // _CHUNK


def _sc_gather_body(w_hbm, idx_hbm, out_hbm, idx_v, rows_v, sem):
    wid = lax.axis_index("s") * 2 + lax.axis_index("c")
    base = wid * _ROWS_PER_W
    pltpu.sync_copy(idx_hbm.at[wid], idx_v)
    cps = [
        pltpu.async_copy(w_hbm.at[idx_v.at[j]],
                         rows_v.at[pl.ds(j * _CHUNK, _CHUNK)], sem)
        for j in range(_NCHUNK)
    ]
    for cp in cps:
        cp.wait()
    pltpu.sync_copy(rows_v, out_hbm.at[pl.ds(base, _ROWS_PER_W)])


def _sc_gather(W, idx3):
    gk = pl.kernel(
        _sc_gather_body,
        out_type=jax.ShapeDtypeStruct((_NW * _ROWS_PER_W, _E_DIM),
                                      jnp.float32),
        mesh=plsc.VectorSubcoreMesh(core_axis_name="c",
                                    subcore_axis_name="s"),
        scratch_types=[
            pltpu.VMEM((_NCHUNK, _CHUNK), jnp.int32),
            pltpu.VMEM((_ROWS_PER_W, _E_DIM), jnp.float32),
            pltpu.SemaphoreType.DMA,
        ],
        compiler_params=pltpu.CompilerParams(use_tc_tiling_on_sc=False),
    )
    return gk(W, idx3)


def kernel(z, W):
    b, c, h, w = z.shape
    n = b * h * w
    zr = z.reshape(b, c, h * w)
    Wb2 = (2.0 * W).astype(jnp.bfloat16)

    idx2, loss_sum = _tc_argmin(zr, Wb2)
    idx_flat = idx2.reshape(n)
    idx3 = idx2.reshape(_NW, _NCHUNK, _CHUNK)

    zq_flat = _sc_gather(W, idx3)

    zq = zq_flat.reshape(b, h, w, c)
    z_q_out = jnp.transpose(zq, (0, 3, 1, 2))

    m = loss_sum[0, 0] / jnp.float32(n * c)
    loss = m + _BETA * m

    z_indices = idx_flat.reshape(b, 1, h, w)
    return (z_q_out, loss, idx_flat, z_indices)
